# Initial kernel scaffold; baseline (speedup 1.0000x reference)
import jax
import jax.numpy as jnp
from jax.experimental import pallas as pl


def kernel(x, pos, Wqkv, W1, b1, W2, b2, A1, ab1, A2, ab2):
    raise NotImplementedError("write your pallas kernel here")


# speedup vs baseline: 1.6968x; 1.6968x over previous
import functools

import jax
import jax.numpy as jnp
from jax.experimental import pallas as pl
from jax.experimental.pallas import tpu as pltpu

_KNN = 16
_BIG = 3.0e38


def _ptl_kernel(x_ref, pos_ref, wq_ref, wk_ref, wv_ref, w1_ref, b1_ref,
                w2_ref, b2_ref, a1_ref, ab1_ref, a2_ref, ab2_ref, out_ref,
                th_ref, li_ref, m_ref, s_ref, acc_ref):
    t = pl.program_id(1)
    n = x_ref.shape[1]
    first = t == 0

    f32 = jnp.float32
    dot = functools.partial(jax.lax.dot_general, preferred_element_type=f32,
                            precision=jax.lax.Precision.HIGHEST)
    pos = pos_ref[0]
    x = x_ref[0]

    q = dot(x, wq_ref[...], (((1,), (0,)), ((), ())))
    k = dot(x, wk_ref[...], (((1,), (0,)), ((), ())))
    v = dot(x, wv_ref[...], (((1,), (0,)), ((), ())))

    pnj = jnp.sum(pos * pos, axis=1, keepdims=True)
    ones = jnp.ones(pnj.shape, f32)
    dots = dot(pos, pos, (((1,), (1,)), ((), ())))
    pnn = dot(ones, pnj, (((1,), (1,)), ((), ())))
    key = pnn - 2.0 * dots

    th = jnp.where(first, -_BIG, th_ref[...])
    li = jnp.where(first, -1, li_ref[...])
    iota_j = jax.lax.broadcasted_iota(jnp.int32, key.shape, 1)
    eligible = (key > th) | ((key == th) & (iota_j > li))
    keyx = jnp.where(eligible, key, _BIG)
    rmin = jnp.min(keyx, axis=1, keepdims=True)
    cand = jnp.where(keyx == rmin, iota_j, n)
    amin = jnp.min(cand, axis=1, keepdims=True)
    sel = iota_j == amin
    oh = sel.astype(f32)
    th_ref[...] = rmin
    li_ref[...] = amin

    k_g = dot(oh, k, (((1,), (0,)), ((), ())))
    v_g = dot(oh, v, (((1,), (0,)), ((), ())))
    p_g = dot(oh, pos, (((1,), (0,)), ((), ())))

    rel = pos - p_g
    h = jax.nn.relu(dot(rel, w1_ref[...], (((1,), (0,)), ((), ())))
                    + b1_ref[...])
    pe = dot(h, w2_ref[...], (((1,), (0,)), ((), ()))) + b2_ref[...]

    s_in = q - k_g + pe
    h2 = jax.nn.relu(dot(s_in, a1_ref[...], (((1,), (0,)), ((), ())))
                     + ab1_ref[...])
    sim = dot(h2, a2_ref[...], (((1,), (0,)), ((), ()))) + ab2_ref[...]
    v2 = v_g + pe

    m = jnp.where(first, -_BIG, m_ref[...])
    s = jnp.where(first, 0.0, s_ref[...])
    acc = jnp.where(first, 0.0, acc_ref[...])
    m_new = jnp.maximum(m, sim)
    c = jnp.exp(m - m_new)
    p = jnp.exp(sim - m_new)
    s_new = s * c + p
    acc_new = acc * c + p * v2
    m_ref[...] = m_new
    s_ref[...] = s_new
    acc_ref[...] = acc_new

    out_ref[0] = acc_new / s_new


def kernel(x, pos, Wqkv, W1, b1, W2, b2, A1, ab1, A2, ab2):
    b, n, d = x.shape
    wq = Wqkv[:, :d]
    wk = Wqkv[:, d:2 * d]
    wv = Wqkv[:, 2 * d:]

    full = lambda a: pl.BlockSpec(a.shape, lambda i, t: (0,) * a.ndim)
    return pl.pallas_call(
        _ptl_kernel,
        grid=(b, _KNN),
        in_specs=[
            pl.BlockSpec((1, n, d), lambda i, t: (i, 0, 0)),
            pl.BlockSpec((1, n, 3), lambda i, t: (i, 0, 0)),
            full(wq), full(wk), full(wv),
            full(W1), pl.BlockSpec((1, b1.shape[0]), lambda i, t: (0, 0)), full(W2), pl.BlockSpec((1, b2.shape[0]), lambda i, t: (0, 0)),
            full(A1), pl.BlockSpec((1, ab1.shape[0]), lambda i, t: (0, 0)), full(A2), pl.BlockSpec((1, ab2.shape[0]), lambda i, t: (0, 0)),
        ],
        out_specs=pl.BlockSpec((1, n, d), lambda i, t: (i, 0, 0)),
        out_shape=jax.ShapeDtypeStruct((b, n, d), jnp.float32),
        scratch_shapes=[
            pltpu.VMEM((n, 1), jnp.float32),
            pltpu.VMEM((n, 1), jnp.int32),
            pltpu.VMEM((n, d), jnp.float32),
            pltpu.VMEM((n, d), jnp.float32),
            pltpu.VMEM((n, d), jnp.float32),
        ],
        compiler_params=pltpu.CompilerParams(
            dimension_semantics=("arbitrary", "arbitrary"),
        ),
    )(x, pos, wq, wk, wv, W1, b1.reshape(1, -1), W2, b2.reshape(1, -1), A1, ab1.reshape(1, -1), A2, ab2.reshape(1, -1))


# speedup vs baseline: 2.0834x; 1.2278x over previous
import functools

import jax
import jax.numpy as jnp
from jax.experimental import pallas as pl
from jax.experimental.pallas import tpu as pltpu

_KNN = 16
_BIG = 3.0e38


def _setup_kernel(x_ref, pos_ref, wqkv_ref, qkv_ref, key_ref):
    f32 = jnp.float32
    pos = pos_ref[0]
    x = x_ref[0]

    qkv_ref[0] = jax.lax.dot_general(
        x, wqkv_ref[...], (((1,), (0,)), ((), ())),
        preferred_element_type=f32, precision=jax.lax.Precision.HIGHEST)

    dot_hi = functools.partial(jax.lax.dot_general, preferred_element_type=f32,
                               precision=jax.lax.Precision.HIGHEST)
    pnj = jnp.sum(pos * pos, axis=1, keepdims=True)
    ones = jnp.ones(pnj.shape, f32)
    dots = dot_hi(pos, pos, (((1,), (1,)), ((), ())))
    pnn = dot_hi(ones, pnj, (((1,), (1,)), ((), ())))
    key_ref[0] = pnn - 2.0 * dots


def _rounds_kernel(key_in_ref, q_ref, k_ref, v_ref, pos_ref, w1_ref, b1_ref,
                   w2_ref, b2_ref, a1_ref, ab1_ref, a2_ref, ab2_ref, out_ref,
                   th_ref, li_ref, m_ref, s_ref, acc_ref):
    t = pl.program_id(1)
    n = key_in_ref.shape[1]
    first = t == 0

    f32 = jnp.float32
    dot = functools.partial(jax.lax.dot_general, preferred_element_type=f32,
                            precision=jax.lax.Precision.HIGHEST)
    pos = pos_ref[0]
    key = key_in_ref[0]

    th = jnp.where(first, -_BIG, th_ref[...])
    li = jnp.where(first, -1, li_ref[...])
    iota_j = jax.lax.broadcasted_iota(jnp.int32, key.shape, 1)
    eligible = (key > th) | ((key == th) & (iota_j > li))
    keyx = jnp.where(eligible, key, _BIG)
    rmin = jnp.min(keyx, axis=1, keepdims=True)
    cand = jnp.where(keyx == rmin, iota_j, n)
    amin = jnp.min(cand, axis=1, keepdims=True)
    sel = iota_j == amin
    oh = sel.astype(f32)
    th_ref[...] = rmin
    li_ref[...] = amin

    k_g = dot(oh, k_ref[0], (((1,), (0,)), ((), ())))
    v_g = dot(oh, v_ref[0], (((1,), (0,)), ((), ())))
    p_g = dot(oh, pos, (((1,), (0,)), ((), ())))

    rel = pos - p_g
    h = jax.nn.relu(dot(rel, w1_ref[...], (((1,), (0,)), ((), ())))
                    + b1_ref[...])
    pe = dot(h, w2_ref[...], (((1,), (0,)), ((), ()))) + b2_ref[...]

    s_in = q_ref[0] - k_g + pe
    h2 = jax.nn.relu(dot(s_in, a1_ref[...], (((1,), (0,)), ((), ())))
                     + ab1_ref[...])
    sim = dot(h2, a2_ref[...], (((1,), (0,)), ((), ()))) + ab2_ref[...]
    v2 = v_g + pe

    m = jnp.where(first, -_BIG, m_ref[...])
    s = jnp.where(first, 0.0, s_ref[...])
    acc = jnp.where(first, 0.0, acc_ref[...])
    m_new = jnp.maximum(m, sim)
    c = jnp.exp(m - m_new)
    p = jnp.exp(sim - m_new)
    s_new = s * c + p
    acc_new = acc * c + p * v2
    m_ref[...] = m_new
    s_ref[...] = s_new
    acc_ref[...] = acc_new

    out_ref[0] = acc_new / s_new


def kernel(x, pos, Wqkv, W1, b1, W2, b2, A1, ab1, A2, ab2):
    b, n, d = x.shape

    qkv, key = pl.pallas_call(
        _setup_kernel,
        grid=(b,),
        in_specs=[
            pl.BlockSpec((1, n, d), lambda i: (i, 0, 0)),
            pl.BlockSpec((1, n, 3), lambda i: (i, 0, 0)),
            pl.BlockSpec(Wqkv.shape, lambda i: (0, 0)),
        ],
        out_specs=[
            pl.BlockSpec((1, n, 3 * d), lambda i: (i, 0, 0)),
            pl.BlockSpec((1, n, n), lambda i: (i, 0, 0)),
        ],
        out_shape=[
            jax.ShapeDtypeStruct((b, n, 3 * d), jnp.float32),
            jax.ShapeDtypeStruct((b, n, n), jnp.float32),
        ],
    )(x, pos, Wqkv)
    q = qkv[:, :, :d]
    k = qkv[:, :, d:2 * d]
    v = qkv[:, :, 2 * d:]

    full = lambda a: pl.BlockSpec(a.shape, lambda i, t: (0,) * a.ndim)
    row = lambda a: pl.BlockSpec((1, a.shape[0]), lambda i, t: (0, 0))
    bat = lambda w: pl.BlockSpec((1, n, w), lambda i, t: (i, 0, 0))
    return pl.pallas_call(
        _rounds_kernel,
        grid=(b, _KNN),
        in_specs=[
            bat(n), bat(d), bat(d), bat(d), bat(3),
            full(W1), row(b1), full(W2), row(b2),
            full(A1), row(ab1), full(A2), row(ab2),
        ],
        out_specs=pl.BlockSpec((1, n, d), lambda i, t: (i, 0, 0)),
        out_shape=jax.ShapeDtypeStruct((b, n, d), jnp.float32),
        scratch_shapes=[
            pltpu.VMEM((n, 1), jnp.float32),
            pltpu.VMEM((n, 1), jnp.int32),
            pltpu.VMEM((n, d), jnp.float32),
            pltpu.VMEM((n, d), jnp.float32),
            pltpu.VMEM((n, d), jnp.float32),
        ],
        compiler_params=pltpu.CompilerParams(
            dimension_semantics=("arbitrary", "arbitrary"),
        ),
    )(key, q, k, v, pos, W1, b1.reshape(1, -1), W2, b2.reshape(1, -1),
      A1, ab1.reshape(1, -1), A2, ab2.reshape(1, -1))


# speedup vs baseline: 3.7100x; 1.7808x over previous
import functools

import jax
import jax.numpy as jnp
from jax.experimental import pallas as pl
from jax.experimental.pallas import tpu as pltpu

_KNN = 16
_BIG = 3.0e38


def _setup_kernel(x_ref, pos_ref, wqkv_ref, w1_ref, q_ref, pw_ref, key_ref,
                  thi_ref, tmid_ref, tlo_ref):
    f32 = jnp.float32
    bf16 = jnp.bfloat16
    dot_hi = functools.partial(jax.lax.dot_general, preferred_element_type=f32,
                               precision=jax.lax.Precision.HIGHEST)
    pos = pos_ref[0]
    x = x_ref[0]
    d = x.shape[1]

    qkv = dot_hi(x, wqkv_ref[...], (((1,), (0,)), ((), ())))
    q_ref[0] = qkv[:, :d]
    pw = dot_hi(pos, w1_ref[...], (((1,), (0,)), ((), ())))
    pw_ref[0] = pw

    pnj = jnp.sum(pos * pos, axis=1, keepdims=True)
    ones = jnp.ones(pnj.shape, f32)
    dots = dot_hi(pos, pos, (((1,), (1,)), ((), ())))
    pnn = dot_hi(ones, pnj, (((1,), (1,)), ((), ())))
    key_ref[0] = pnn - 2.0 * dots

    tbl = jnp.concatenate([qkv[:, d:], pw], axis=1)
    t_hi = tbl.astype(bf16)
    r1 = tbl - t_hi.astype(f32)
    t_mid = r1.astype(bf16)
    r2 = r1 - t_mid.astype(f32)
    thi_ref[0] = t_hi
    tmid_ref[0] = t_mid
    tlo_ref[0] = r2.astype(bf16)


def _rounds_kernel(key_in_ref, q_ref, pw_ref, thi_ref, tmid_ref, tlo_ref,
                   b1_ref, w2_ref, b2_ref, a1_ref, ab1_ref, a2_ref, ab2_ref,
                   out_ref, th_ref, li_ref, m_ref, s_ref, acc_ref):
    t = pl.program_id(1)
    n = key_in_ref.shape[1]
    d = q_ref.shape[2]
    first = t == 0

    f32 = jnp.float32
    dot = functools.partial(jax.lax.dot_general, preferred_element_type=f32,
                            precision=jax.lax.Precision.HIGHEST)
    dot_bf = functools.partial(jax.lax.dot_general, preferred_element_type=f32)
    key = key_in_ref[0]

    th = jnp.where(first, -_BIG, th_ref[...])
    li = jnp.where(first, -1, li_ref[...])
    iota_j = jax.lax.broadcasted_iota(jnp.int32, key.shape, 1)
    eligible = (key > th) | ((key == th) & (iota_j > li))
    keyx = jnp.where(eligible, key, _BIG)
    rmin = jnp.min(keyx, axis=1, keepdims=True)
    cand = jnp.where(keyx == rmin, iota_j, n)
    amin = jnp.min(cand, axis=1, keepdims=True)
    sel = iota_j == amin
    oh = sel.astype(jnp.bfloat16)
    th_ref[...] = rmin
    li_ref[...] = amin

    cdims = (((1,), (0,)), ((), ()))
    g = (dot_bf(oh, thi_ref[0], cdims) + dot_bf(oh, tmid_ref[0], cdims)
         + dot_bf(oh, tlo_ref[0], cdims))
    k_g = g[:, :d]
    v_g = g[:, d:2 * d]
    pw_g = g[:, 2 * d:]

    h = jax.nn.relu(pw_ref[0] - pw_g + b1_ref[...])
    pe = dot(h, w2_ref[...], cdims) + b2_ref[...]

    s_in = q_ref[0] - k_g + pe
    h2 = jax.nn.relu(dot(s_in, a1_ref[...], cdims) + ab1_ref[...])
    sim = dot(h2, a2_ref[...], cdims) + ab2_ref[...]
    v2 = v_g + pe

    m = jnp.where(first, -_BIG, m_ref[...])
    s = jnp.where(first, 0.0, s_ref[...])
    acc = jnp.where(first, 0.0, acc_ref[...])
    m_new = jnp.maximum(m, sim)
    c = jnp.exp(m - m_new)
    p = jnp.exp(sim - m_new)
    s_new = s * c + p
    acc_new = acc * c + p * v2
    m_ref[...] = m_new
    s_ref[...] = s_new
    acc_ref[...] = acc_new

    out_ref[0] = acc_new / s_new


def kernel(x, pos, Wqkv, W1, b1, W2, b2, A1, ab1, A2, ab2):
    b, n, d = x.shape

    q, pw, key, t_hi, t_mid, t_lo = pl.pallas_call(
        _setup_kernel,
        grid=(b,),
        in_specs=[
            pl.BlockSpec((1, n, d), lambda i: (i, 0, 0)),
            pl.BlockSpec((1, n, 3), lambda i: (i, 0, 0)),
            pl.BlockSpec(Wqkv.shape, lambda i: (0, 0)),
            pl.BlockSpec(W1.shape, lambda i: (0, 0)),
        ],
        out_specs=[
            pl.BlockSpec((1, n, d), lambda i: (i, 0, 0)),
            pl.BlockSpec((1, n, d), lambda i: (i, 0, 0)),
            pl.BlockSpec((1, n, n), lambda i: (i, 0, 0)),
            pl.BlockSpec((1, n, 3 * d), lambda i: (i, 0, 0)),
            pl.BlockSpec((1, n, 3 * d), lambda i: (i, 0, 0)),
            pl.BlockSpec((1, n, 3 * d), lambda i: (i, 0, 0)),
        ],
        out_shape=[
            jax.ShapeDtypeStruct((b, n, d), jnp.float32),
            jax.ShapeDtypeStruct((b, n, d), jnp.float32),
            jax.ShapeDtypeStruct((b, n, n), jnp.float32),
            jax.ShapeDtypeStruct((b, n, 3 * d), jnp.bfloat16),
            jax.ShapeDtypeStruct((b, n, 3 * d), jnp.bfloat16),
            jax.ShapeDtypeStruct((b, n, 3 * d), jnp.bfloat16),
        ],
    )(x, pos, Wqkv, W1)

    full = lambda a: pl.BlockSpec(a.shape, lambda i, t: (0,) * a.ndim)
    row = lambda a: pl.BlockSpec((1, a.shape[0]), lambda i, t: (0, 0))
    bat = lambda w: pl.BlockSpec((1, n, w), lambda i, t: (i, 0, 0))
    return pl.pallas_call(
        _rounds_kernel,
        grid=(b, _KNN),
        in_specs=[
            bat(n), bat(d), bat(d), bat(3 * d), bat(3 * d), bat(3 * d),
            row(b1), full(W2), row(b2),
            full(A1), row(ab1), full(A2), row(ab2),
        ],
        out_specs=pl.BlockSpec((1, n, d), lambda i, t: (i, 0, 0)),
        out_shape=jax.ShapeDtypeStruct((b, n, d), jnp.float32),
        scratch_shapes=[
            pltpu.VMEM((n, 1), jnp.float32),
            pltpu.VMEM((n, 1), jnp.int32),
            pltpu.VMEM((n, d), jnp.float32),
            pltpu.VMEM((n, d), jnp.float32),
            pltpu.VMEM((n, d), jnp.float32),
        ],
        compiler_params=pltpu.CompilerParams(
            dimension_semantics=("arbitrary", "arbitrary"),
        ),
    )(key, q, pw, t_hi, t_mid, t_lo, b1.reshape(1, -1), W2,
      b2.reshape(1, -1), A1, ab1.reshape(1, -1), A2, ab2.reshape(1, -1))
